# Initial kernel scaffold; baseline (speedup 1.0000x reference)
import jax
import jax.numpy as jnp
from jax.experimental import pallas as pl


def kernel(adj, embeds):
    raise NotImplementedError("write your pallas kernel here")


# speedup vs baseline: 1.0229x; 1.0229x over previous
import jax
import jax.numpy as jnp
from jax.experimental import pallas as pl
from jax.experimental.pallas import tpu as pltpu

_NEG_SLOPE = 0.5
_M = 10000
_K = 2048
_F = 128
_BM = 1000
_T = _M // _BM


def _leaky(x):
    return jnp.where(x >= 0, x, _NEG_SLOPE * x)


def _fused(adj_ref, emb_ref, out_ref, hacc_ref, hact_ref):
    p = pl.program_id(0)
    t = pl.program_id(1)

    @pl.when(p == 0)
    def _phase1():
        a = adj_ref[...].astype(jnp.bfloat16)
        e = emb_ref[...].astype(jnp.bfloat16)
        part = jax.lax.dot_general(
            a, e, (((0,), (0,)), ((), ())),
            preferred_element_type=jnp.float32)

        @pl.when(t == 0)
        def _init():
            hacc_ref[...] = part

        @pl.when(t != 0)
        def _accum():
            hacc_ref[...] += part

    @pl.when(p == 1)
    def _phase2():
        @pl.when(t == 0)
        def _activate():
            hact_ref[...] = _leaky(hacc_ref[...]).astype(jnp.bfloat16)

        a = adj_ref[...].astype(jnp.bfloat16)
        o = jax.lax.dot_general(
            a, hact_ref[...], (((1,), (0,)), ((), ())),
            preferred_element_type=jnp.float32)
        out_ref[...] = _leaky(o)


def kernel(adj, embeds):
    return pl.pallas_call(
        _fused,
        grid=(2, _T),
        in_specs=[
            pl.BlockSpec((_BM, _K), lambda p, t: (t, 0)),
            pl.BlockSpec((_BM, _F), lambda p, t: (jnp.where(p == 0, t, 0), 0)),
        ],
        out_specs=pl.BlockSpec((_BM, _F), lambda p, t: (jnp.where(p == 1, t, 0), 0)),
        out_shape=jax.ShapeDtypeStruct((_M, _F), jnp.float32),
        scratch_shapes=[
            pltpu.VMEM((_K, _F), jnp.float32),
            pltpu.VMEM((_K, _F), jnp.bfloat16),
        ],
        compiler_params=pltpu.CompilerParams(
            dimension_semantics=("arbitrary", "arbitrary"),
        ),
    )(adj, embeds)
